# Initial kernel scaffold; baseline (speedup 1.0000x reference)
import jax
import jax.numpy as jnp
from jax import lax
from jax.experimental import pallas as pl
from jax.experimental.pallas import tpu as pltpu
from jax.experimental.pallas import tpu_sc as plsc

NUM_EMB = 1000000
G_DIM = 64
R_DIM = 64
OUT_DIM = G_DIM + R_DIM
BATCH = 4096
NB_WORDS = 50
B_TOTAL = BATCH * NB_WORDS

NC = 2
NS = 16
NW = NC * NS
B_PER_W = B_TOTAL // NW
CHUNK = 128
N_CHUNKS = B_PER_W // CHUNK


def _emb_body(x_hbm, g_hbm, r_hbm, out_hbm, idx_v, gbuf, rbuf,
              gsem, rsem):
    wid = lax.axis_index("s") * NC + lax.axis_index("c")
    base = wid * B_PER_W

    def chunk_body(c, _):
        off = base + c * CHUNK
        pltpu.sync_copy(x_hbm.at[pl.ds(off, CHUNK)], idx_v)
        cg = pltpu.async_copy(g_hbm.at[idx_v], gbuf, gsem)
        cr = pltpu.async_copy(r_hbm.at[idx_v], rbuf, rsem)
        cg.wait()
        pltpu.sync_copy(gbuf, out_hbm.at[pl.ds(off, CHUNK), pl.ds(0, G_DIM)])
        cr.wait()
        pltpu.sync_copy(rbuf, out_hbm.at[pl.ds(off, CHUNK), pl.ds(G_DIM, R_DIM)])
        return 0

    lax.fori_loop(0, N_CHUNKS, chunk_body, 0)


@jax.jit
def _emb_call(x_flat, glove_weight, rand_weight):
    kern = pl.kernel(
        _emb_body,
        out_type=jax.ShapeDtypeStruct((B_TOTAL, OUT_DIM), jnp.float32),
        mesh=plsc.VectorSubcoreMesh(core_axis_name="c", subcore_axis_name="s"),
        scratch_types=[
            pltpu.VMEM((CHUNK,), jnp.int32),
            pltpu.VMEM((CHUNK, G_DIM), jnp.float32),
            pltpu.VMEM((CHUNK, R_DIM), jnp.float32),
            pltpu.SemaphoreType.DMA,
            pltpu.SemaphoreType.DMA,
        ],
    )
    return kern(x_flat, glove_weight, rand_weight)


def kernel(x, glove_weight, rand_weight):
    x_flat = x.reshape(B_TOTAL)
    out = _emb_call(x_flat, glove_weight, rand_weight)
    return out.reshape(BATCH, NB_WORDS, OUT_DIM)


# speedup vs baseline: 1.2036x; 1.2036x over previous
import jax
import jax.numpy as jnp
from jax import lax
from jax.experimental import pallas as pl
from jax.experimental.pallas import tpu as pltpu
from jax.experimental.pallas import tpu_sc as plsc

NUM_EMB = 1000000
G_DIM = 64
R_DIM = 64
OUT_DIM = G_DIM + R_DIM
BATCH = 4096
NB_WORDS = 50
B_TOTAL = BATCH * NB_WORDS

NC = 2
NS = 16
NW = NC * NS
B_PER_W = B_TOTAL // NW
CHUNK = 128
N_CHUNKS = B_PER_W // CHUNK


def _emb_body(x_hbm, g_hbm, r_hbm, out_hbm, idx_v, gbuf, rbuf, comb,
              gsem, rsem):
    wid = lax.axis_index("s") * NC + lax.axis_index("c")
    base = wid * B_PER_W

    def chunk_body(c, _):
        off = base + c * CHUNK
        pltpu.sync_copy(x_hbm.at[pl.ds(off, CHUNK)], idx_v)
        cg = pltpu.async_copy(g_hbm.at[idx_v], gbuf, gsem)
        cr = pltpu.async_copy(r_hbm.at[idx_v], rbuf, rsem)
        cg.wait()
        cr.wait()

        def row_body(i, _):
            for k in range(0, G_DIM, 16):
                comb[i, pl.ds(k, 16)] = gbuf[i, pl.ds(k, 16)]
            for k in range(0, R_DIM, 16):
                comb[i, pl.ds(G_DIM + k, 16)] = rbuf[i, pl.ds(k, 16)]
            return 0

        lax.fori_loop(0, CHUNK, row_body, 0)
        pltpu.sync_copy(comb, out_hbm.at[pl.ds(off, CHUNK), :])
        return 0

    lax.fori_loop(0, N_CHUNKS, chunk_body, 0)


def _emb_call(x_flat, glove_pad, rand_pad):
    kern = pl.kernel(
        _emb_body,
        out_type=jax.ShapeDtypeStruct((B_TOTAL, OUT_DIM), jnp.float32),
        mesh=plsc.VectorSubcoreMesh(core_axis_name="c", subcore_axis_name="s"),
        scratch_types=[
            pltpu.VMEM((CHUNK,), jnp.int32),
            pltpu.VMEM((CHUNK, 128), jnp.float32),
            pltpu.VMEM((CHUNK, 128), jnp.float32),
            pltpu.VMEM((CHUNK, OUT_DIM), jnp.float32),
            pltpu.SemaphoreType.DMA,
            pltpu.SemaphoreType.DMA,
        ],
    )
    return kern(x_flat, glove_pad, rand_pad)


def kernel(x, glove_weight, rand_weight):
    x_flat = x.reshape(B_TOTAL)
    glove_pad = jnp.pad(glove_weight, ((0, 0), (0, 128 - G_DIM)))
    rand_pad = jnp.pad(rand_weight, ((0, 0), (0, 128 - R_DIM)))
    out = _emb_call(x_flat, glove_pad, rand_pad)
    return out.reshape(BATCH, NB_WORDS, OUT_DIM)
